# Initial kernel scaffold; baseline (speedup 1.0000x reference)
import jax
import jax.numpy as jnp
from jax.experimental import pallas as pl


def kernel(src, emb, W, b):
    raise NotImplementedError("write your pallas kernel here")


# speedup vs baseline: 91.8981x; 91.8981x over previous
import functools

import jax
import jax.numpy as jnp
from jax import lax
from jax.experimental import pallas as pl
from jax.experimental.pallas import tpu as pltpu
from jax.experimental.pallas import tpu_sc as plsc

_MSP = 800
_T = 102
_D = 256
_PAD = 799
_J = 4
_B = 16384

_NC = 2
_NS = 16
_NW = _NC * _NS
_L = 16

_CHUNK = _B // (_NW // _J)
_SUB = 256
_NSUB = _CHUNK // _SUB
_NG = _SUB // _L


def _tc_table_body(emb_ref, wd_ref, b_ref, out_ref):
    row = lax.broadcasted_iota(jnp.int32, (_MSP, _D), 0)
    embz = jnp.where(row == _PAD, 0.0, emb_ref[...])
    out_ref[...] = (
        jnp.dot(embz, wd_ref[...], preferred_element_type=jnp.float32) + b_ref[...]
    )


def _build_table(emb, W, b):
    wd = W.reshape(_J, _T, _D).transpose(2, 1, 0).reshape(_D, _T * _J)
    bpad = jnp.pad(b, (0, _T * _J - _J)).reshape(1, _T * _J)
    p2 = pl.pallas_call(
        _tc_table_body,
        out_shape=jax.ShapeDtypeStruct((_MSP, _T * _J), jnp.float32),
    )(emb, wd, bpad)
    return p2.reshape(_MSP, _T, _J).transpose(2, 1, 0).reshape(_J, _T * _MSP)


def _sc_body(table_hbm, srcT_hbm, out_hbm, tab_v, src_v, acc_v):
    wid = lax.axis_index("s") * _NC + lax.axis_index("c")
    j = wid % _J
    c = wid // _J
    b0 = c * _CHUNK
    pltpu.sync_copy(table_hbm.at[j], tab_v)

    def sub_body(s, carry):
        sb = b0 + s * _SUB
        pltpu.sync_copy(srcT_hbm.at[:, pl.ds(sb, _SUB)], src_v)

        def g_body(g, carry2):
            def t_body(t, acc):
                iv = src_v[t, pl.ds(g * _L, _L)]
                idx = iv + t * _MSP
                return acc + plsc.load_gather(tab_v, [idx])

            acc = lax.fori_loop(0, _T, t_body, jnp.zeros((_L,), jnp.float32))
            acc_v[pl.ds(g * _L, _L)] = acc
            return carry2

        lax.fori_loop(0, _NG, g_body, 0)
        pltpu.sync_copy(acc_v, out_hbm.at[j, pl.ds(sb, _SUB)])
        return carry

    lax.fori_loop(0, _NSUB, sub_body, 0)


_sc_lookup = functools.partial(
    pl.kernel,
    out_type=jax.ShapeDtypeStruct((_J, _B), jnp.float32),
    mesh=plsc.VectorSubcoreMesh(core_axis_name="c", subcore_axis_name="s"),
    compiler_params=pltpu.CompilerParams(needs_layout_passes=False),
    scratch_types=[
        pltpu.VMEM((_T * _MSP,), jnp.float32),
        pltpu.VMEM((_T, _SUB), jnp.int32),
        pltpu.VMEM((_SUB,), jnp.float32),
    ],
)(_sc_body)


def kernel(src, emb, W, b):
    table = _build_table(emb, W, b)
    srcT = src.T
    outT = _sc_lookup(table, srcT)
    return outT.T


# speedup vs baseline: 167.1146x; 1.8185x over previous
import functools

import jax
import jax.numpy as jnp
from jax import lax
from jax.experimental import pallas as pl
from jax.experimental.pallas import tpu as pltpu
from jax.experimental.pallas import tpu_sc as plsc

_MSP = 800
_T = 102
_D = 256
_PAD = 799
_J = 4
_B = 16384

_NC = 2
_NS = 16
_NW = _NC * _NS
_L = 16

_CHUNK = _B // (_NW // _J)
_SUB = 128
_NSUB = _CHUNK // _SUB
_NG = _SUB // _L


def _tc_table_body(emb_ref, wd_ref, b_ref, out_ref):
    row = lax.broadcasted_iota(jnp.int32, (_MSP, _D), 0)
    embz = jnp.where(row == _PAD, 0.0, emb_ref[...])
    out_ref[...] = (
        jnp.dot(embz, wd_ref[...], preferred_element_type=jnp.float32) + b_ref[...]
    )


def _build_table(emb, W, b):
    wd = W.reshape(_J, _T, _D).transpose(2, 1, 0).reshape(_D, _T * _J)
    bpad = jnp.pad(b, (0, _T * _J - _J)).reshape(1, _T * _J)
    p2 = pl.pallas_call(
        _tc_table_body,
        out_shape=jax.ShapeDtypeStruct((_MSP, _T * _J), jnp.float32),
    )(emb, wd, bpad)
    return p2.reshape(_MSP, _T, _J).transpose(2, 1, 0).reshape(_J, _T * _MSP)


def _sc_body(table_hbm, srcT_hbm, out_hbm, tab_v, src_v0, src_v1, acc_v, sem0, sem1):
    wid = lax.axis_index("s") * _NC + lax.axis_index("c")
    j = wid % _J
    c = wid // _J
    b0 = c * _CHUNK
    pltpu.sync_copy(table_hbm.at[j], tab_v)

    sems = (sem0, sem1)
    srcs = (src_v0, src_v1)

    def start(s):
        return pltpu.async_copy(
            srcT_hbm.at[:, pl.ds(b0 + s * _SUB, _SUB)], srcs[s % 2], sems[s % 2]
        )

    pending = start(0)
    for s in range(_NSUB):
        buf = s % 2
        nxt = start(s + 1) if s + 1 < _NSUB else None
        pending.wait()
        pending = nxt

        sbuf = srcs[buf]

        def t_body(t, accs):
            toff = t * _MSP
            out = []
            for g in range(_NG):
                iv = sbuf[t, pl.ds(g * _L, _L)]
                out.append(accs[g] + plsc.load_gather(tab_v, [iv + toff]))
            return tuple(out)

        accs = lax.fori_loop(
            0, _T, t_body, tuple(jnp.zeros((_L,), jnp.float32) for _ in range(_NG))
        )
        for g in range(_NG):
            acc_v[pl.ds(s * _SUB + g * _L, _L)] = accs[g]

    pltpu.sync_copy(acc_v, out_hbm.at[j, pl.ds(b0, _CHUNK)])


_sc_lookup = functools.partial(
    pl.kernel,
    out_type=jax.ShapeDtypeStruct((_J, _B), jnp.float32),
    mesh=plsc.VectorSubcoreMesh(core_axis_name="c", subcore_axis_name="s"),
    compiler_params=pltpu.CompilerParams(needs_layout_passes=False),
    scratch_types=[
        pltpu.VMEM((_T * _MSP,), jnp.float32),
        pltpu.VMEM((_T, _SUB), jnp.int32),
        pltpu.VMEM((_T, _SUB), jnp.int32),
        pltpu.VMEM((_CHUNK,), jnp.float32),
        pltpu.SemaphoreType.DMA,
        pltpu.SemaphoreType.DMA,
    ],
)(_sc_body)


def kernel(src, emb, W, b):
    table = _build_table(emb, W, b)
    srcT = src.T
    outT = _sc_lookup(table, srcT)
    return outT.T


# speedup vs baseline: 172.3787x; 1.0315x over previous
import functools

import jax
import jax.numpy as jnp
from jax import lax
from jax.experimental import pallas as pl
from jax.experimental.pallas import tpu as pltpu
from jax.experimental.pallas import tpu_sc as plsc

_MSP = 800
_T = 102
_D = 256
_PAD = 799
_J = 4
_B = 16384

_NC = 2
_NS = 16
_NW = _NC * _NS
_L = 16

_NP = _J // 2
_CHUNK = _B // (_NW // _NP)
_SUB = 128
_NSUB = _CHUNK // _SUB
_NG = _SUB // _L


def _tc_table_body(emb_ref, wd_ref, b_ref, out_ref):
    row = lax.broadcasted_iota(jnp.int32, (_MSP, _D), 0)
    embz = jnp.where(row == _PAD, 0.0, emb_ref[...])
    out_ref[...] = (
        jnp.dot(embz, wd_ref[...], preferred_element_type=jnp.float32) + b_ref[...]
    )


def _build_table(emb, W, b):
    wd = W.reshape(_J, _T, _D).transpose(2, 1, 0).reshape(_D, _T * _J)
    bpad = jnp.pad(b, (0, _T * _J - _J)).reshape(1, _T * _J)
    p2 = pl.pallas_call(
        _tc_table_body,
        out_shape=jax.ShapeDtypeStruct((_MSP, _T * _J), jnp.float32),
    )(emb, wd, bpad)
    table = p2.reshape(_MSP, _T, _J).transpose(2, 1, 0).reshape(_J, _T * _MSP)
    bits = lax.bitcast_convert_type(table.astype(jnp.bfloat16), jnp.uint16)
    packed = bits[0::2].astype(jnp.uint32) | (bits[1::2].astype(jnp.uint32) << 16)
    return lax.bitcast_convert_type(packed, jnp.int32)


def _sc_body(table_hbm, srcT_hbm, out_hbm, tab_v, src_v0, src_v1, acc0_v, acc1_v,
             sem0, sem1):
    wid = lax.axis_index("s") * _NC + lax.axis_index("c")
    p = wid % _NP
    c = wid // _NP
    b0 = c * _CHUNK
    pltpu.sync_copy(table_hbm.at[p], tab_v)

    sems = (sem0, sem1)
    srcs = (src_v0, src_v1)
    himask = jnp.full((_L,), -65536, jnp.int32)

    def start(s):
        return pltpu.async_copy(
            srcT_hbm.at[:, pl.ds(b0 + s * _SUB, _SUB)], srcs[s % 2], sems[s % 2]
        )

    pending = start(0)
    for s in range(_NSUB):
        buf = s % 2
        nxt = start(s + 1) if s + 1 < _NSUB else None
        pending.wait()
        pending = nxt

        sbuf = srcs[buf]

        def t_body(t, accs):
            lo_accs, hi_accs = accs
            toff = t * _MSP
            lo_out, hi_out = [], []
            for g in range(_NG):
                iv = sbuf[t, pl.ds(g * _L, _L)]
                w = plsc.load_gather(tab_v, [iv + toff])
                lo = plsc.bitcast(lax.shift_left(w, 16), jnp.float32)
                hi = plsc.bitcast(w & himask, jnp.float32)
                lo_out.append(lo_accs[g] + lo)
                hi_out.append(hi_accs[g] + hi)
            return tuple(lo_out), tuple(hi_out)

        zeros = tuple(jnp.zeros((_L,), jnp.float32) for _ in range(_NG))
        lo_accs, hi_accs = lax.fori_loop(0, _T, t_body, (zeros, zeros))
        for g in range(_NG):
            acc0_v[pl.ds(s * _SUB + g * _L, _L)] = lo_accs[g]
            acc1_v[pl.ds(s * _SUB + g * _L, _L)] = hi_accs[g]

    pltpu.sync_copy(acc0_v, out_hbm.at[2 * p, pl.ds(b0, _CHUNK)])
    pltpu.sync_copy(acc1_v, out_hbm.at[2 * p + 1, pl.ds(b0, _CHUNK)])


_sc_lookup = functools.partial(
    pl.kernel,
    out_type=jax.ShapeDtypeStruct((_J, _B), jnp.float32),
    mesh=plsc.VectorSubcoreMesh(core_axis_name="c", subcore_axis_name="s"),
    compiler_params=pltpu.CompilerParams(needs_layout_passes=False),
    scratch_types=[
        pltpu.VMEM((_T * _MSP,), jnp.int32),
        pltpu.VMEM((_T, _SUB), jnp.int32),
        pltpu.VMEM((_T, _SUB), jnp.int32),
        pltpu.VMEM((_CHUNK,), jnp.float32),
        pltpu.VMEM((_CHUNK,), jnp.float32),
        pltpu.SemaphoreType.DMA,
        pltpu.SemaphoreType.DMA,
    ],
)(_sc_body)


def kernel(src, emb, W, b):
    table = _build_table(emb, W, b)
    srcT = src.T
    outT = _sc_lookup(table, srcT)
    return outT.T


# speedup vs baseline: 202.4418x; 1.1744x over previous
import functools

import jax
import jax.numpy as jnp
from jax import lax
from jax.experimental import pallas as pl
from jax.experimental.pallas import tpu as pltpu
from jax.experimental.pallas import tpu_sc as plsc

_MSP = 800
_T = 102
_D = 256
_PAD = 799
_J = 4
_B = 16384

_NC = 2
_NS = 16
_NW = _NC * _NS
_L = 16

_NP = _J // 2
_CHUNK = _B // (_NW // _NP)
_SUB = 128
_NSUB = _CHUNK // _SUB
_NG = _SUB // _L


_TP = 104


def _tc_table_body(emb_ref, wd_ref, b_ref, out_ref):
    row = lax.broadcasted_iota(jnp.int32, (_MSP, _D), 0)
    embz = jnp.where(row == _PAD, 0.0, emb_ref[...])
    p3 = lax.dot_general(
        wd_ref[...], embz, (((0,), (1,)), ((), ())),
        preferred_element_type=jnp.float32,
    ) + b_ref[...]
    for p in range(_NP):
        lo = p3[p * 2 * _TP : p * 2 * _TP + _TP]
        hi = p3[p * 2 * _TP + _TP : (p + 1) * 2 * _TP]
        lo16 = lax.bitcast_convert_type(lo.astype(jnp.bfloat16), jnp.uint16)
        hi16 = lax.bitcast_convert_type(hi.astype(jnp.bfloat16), jnp.uint16)
        packed = lo16.astype(jnp.uint32) | (hi16.astype(jnp.uint32) << 16)
        out_ref[p] = lax.bitcast_convert_type(packed, jnp.int32)


def _build_table(emb, W, b):
    wd3 = W.reshape(_J, _T, _D).transpose(2, 0, 1)
    wd3 = jnp.pad(wd3, ((0, 0), (0, 0), (0, _TP - _T))).reshape(_D, _J * _TP)
    brow = jnp.zeros((_J * _TP,), jnp.float32).at[jnp.arange(_J) * _TP].set(b)
    packed = pl.pallas_call(
        _tc_table_body,
        out_shape=jax.ShapeDtypeStruct((_NP, _TP, _MSP), jnp.int32),
    )(emb, wd3, brow.reshape(_J * _TP, 1))
    return packed.reshape(_NP, _TP * _MSP)


def _sc_body(table_hbm, srcT_hbm, out_hbm, tab_v, src_v0, src_v1, acc0_v, acc1_v,
             sem0, sem1):
    wid = lax.axis_index("s") * _NC + lax.axis_index("c")
    p = wid % _NP
    c = wid // _NP
    b0 = c * _CHUNK
    pltpu.sync_copy(table_hbm.at[p], tab_v)

    sems = (sem0, sem1)
    srcs = (src_v0, src_v1)
    himask = jnp.full((_L,), -65536, jnp.int32)

    def start(s):
        return pltpu.async_copy(
            srcT_hbm.at[:, pl.ds(b0 + s * _SUB, _SUB)], srcs[s % 2], sems[s % 2]
        )

    pending = start(0)
    for s in range(_NSUB):
        buf = s % 2
        nxt = start(s + 1) if s + 1 < _NSUB else None
        pending.wait()
        pending = nxt

        sbuf = srcs[buf]

        def t_body(t, accs):
            lo_accs, hi_accs = accs
            toff = t * _MSP
            lo_out, hi_out = [], []
            for g in range(_NG):
                iv = sbuf[t, pl.ds(g * _L, _L)]
                w = plsc.load_gather(tab_v, [iv + toff])
                lo = plsc.bitcast(lax.shift_left(w, 16), jnp.float32)
                hi = plsc.bitcast(w & himask, jnp.float32)
                lo_out.append(lo_accs[g] + lo)
                hi_out.append(hi_accs[g] + hi)
            return tuple(lo_out), tuple(hi_out)

        zeros = tuple(jnp.zeros((_L,), jnp.float32) for _ in range(_NG))
        lo_accs, hi_accs = lax.fori_loop(0, _T, t_body, (zeros, zeros))
        for g in range(_NG):
            acc0_v[pl.ds(s * _SUB + g * _L, _L)] = lo_accs[g]
            acc1_v[pl.ds(s * _SUB + g * _L, _L)] = hi_accs[g]

    pltpu.sync_copy(acc0_v, out_hbm.at[2 * p, pl.ds(b0, _CHUNK)])
    pltpu.sync_copy(acc1_v, out_hbm.at[2 * p + 1, pl.ds(b0, _CHUNK)])


_sc_lookup = functools.partial(
    pl.kernel,
    out_type=jax.ShapeDtypeStruct((_J, _B), jnp.float32),
    mesh=plsc.VectorSubcoreMesh(core_axis_name="c", subcore_axis_name="s"),
    compiler_params=pltpu.CompilerParams(needs_layout_passes=False),
    scratch_types=[
        pltpu.VMEM((_TP * _MSP,), jnp.int32),
        pltpu.VMEM((_T, _SUB), jnp.int32),
        pltpu.VMEM((_T, _SUB), jnp.int32),
        pltpu.VMEM((_CHUNK,), jnp.float32),
        pltpu.VMEM((_CHUNK,), jnp.float32),
        pltpu.SemaphoreType.DMA,
        pltpu.SemaphoreType.DMA,
    ],
)(_sc_body)


def kernel(src, emb, W, b):
    table = _build_table(emb, W, b)
    srcT = src.T
    outT = _sc_lookup(table, srcT)
    return outT.T
